# baseline (speedup 1.0000x reference)
import functools

import jax
import jax.numpy as jnp
from jax import lax
from jax.experimental import pallas as pl
from jax.experimental.pallas import tpu as pltpu
from jax.experimental.pallas import tpu_sc as plsc

D_MODEL = 768
LANES = 16
VPR = D_MODEL // LANES
NC = 2
NS = 16
NW = NC * NS
POS_PER_CHUNK = 8
REG_BLOCK = 16
NRING = 4
GAHEAD = NRING - 2
PRING = 2


@functools.partial(jax.jit, static_argnums=(3, 4))
def _embed(xr, token_emb, pos_emb, batch, seq_len):
    s_per_w = seq_len // NW
    n_chunks = s_per_w // POS_PER_CHUNK
    chunk_rows = POS_PER_CHUNK * batch
    mesh = plsc.VectorSubcoreMesh(core_axis_name="c", subcore_axis_name="s")

    @functools.partial(
        pl.kernel,
        out_type=jax.ShapeDtypeStruct((batch, seq_len, D_MODEL), jnp.float32),
        mesh=mesh,
        scratch_types=[
            pltpu.VMEM((n_chunks, chunk_rows), jnp.int32),
            pltpu.VMEM((PRING, POS_PER_CHUNK, D_MODEL), jnp.float32),
            *[pltpu.VMEM((chunk_rows, D_MODEL), jnp.float32)
              for _ in range(NRING)],
            *[pltpu.SemaphoreType.DMA for _ in range(2 * NRING + PRING)],
        ],
    )
    def body(xr_hbm, tok_hbm, pos_hbm, out_hbm, idx_v, pos_ring, *rest):
        bufs = rest[:NRING]
        gsems = rest[NRING:2 * NRING]
        ssems = rest[2 * NRING:3 * NRING]
        psems = rest[3 * NRING:]
        wid = lax.axis_index("s") * NC + lax.axis_index("c")
        s_base = wid * s_per_w

        pltpu.sync_copy(xr_hbm.at[wid], idx_v)

        def start_prefill(c):
            j2 = c % PRING
            return pltpu.async_copy(
                pos_hbm.at[pl.ds(s_base + c * POS_PER_CHUNK, POS_PER_CHUNK)],
                pos_ring.at[j2], psems[j2])

        def start_gather(c):
            j = c % NRING
            return pltpu.async_copy(
                tok_hbm.at[idx_v.at[c]], bufs[j], gsems[j])

        def start_stores(c):
            j = c % NRING
            return [
                pltpu.async_copy(
                    bufs[j].at[pl.ds(b * POS_PER_CHUNK, POS_PER_CHUNK)],
                    out_hbm.at[b, pl.ds(s_base + c * POS_PER_CHUNK,
                                        POS_PER_CHUNK)],
                    ssems[j],
                )
                for b in range(batch)
            ]

        gathers = [None] * NRING
        stores = [None] * NRING
        prefills = [None] * PRING
        for c in range(PRING):
            prefills[c] = start_prefill(c)
        for c in range(GAHEAD):
            gathers[c] = start_gather(c)
        for c in range(n_chunks):
            j = c % NRING
            j2 = c % PRING
            gathers[j].wait()
            prefills[j2].wait()

            buf = bufs[j]

            def add_pos(p, _):
                for t in range(VPR // REG_BLOCK):
                    regs = [
                        pos_ring[j2, p, pl.ds((t * REG_BLOCK + v) * LANES,
                                              LANES)]
                        for v in range(REG_BLOCK)
                    ]
                    for b in range(batch):
                        r = b * POS_PER_CHUNK + p
                        for v in range(REG_BLOCK):
                            sl = pl.ds((t * REG_BLOCK + v) * LANES, LANES)
                            plsc.addupdate(buf.at[r, sl], regs[v])
                return ()

            lax.fori_loop(0, POS_PER_CHUNK, add_pos, (), unroll=False)
            stores[j] = start_stores(c)
            if c + PRING < n_chunks:
                prefills[j2] = start_prefill(c + PRING)
            if c + GAHEAD < n_chunks:
                k = (c + GAHEAD) % NRING
                if stores[k] is not None:
                    for hnd in stores[k]:
                        hnd.wait()
                gathers[k] = start_gather(c + GAHEAD)
        for sset in stores:
            if sset is not None:
                for hnd in sset:
                    hnd.wait()

    return body(xr, token_emb, pos_emb)


def kernel(x, token_emb, pos_emb):
    batch, seq = x.shape
    s_per_w = seq // NW
    n_chunks = s_per_w // POS_PER_CHUNK
    xr = (x.astype(jnp.int32)
          .reshape(batch, NW, n_chunks, POS_PER_CHUNK)
          .transpose(1, 2, 0, 3)
          .reshape(NW, n_chunks, batch * POS_PER_CHUNK))
    return _embed(xr, token_emb, pos_emb, batch, seq)


# speedup vs baseline: 1.1204x; 1.1204x over previous
import functools

import jax
import jax.numpy as jnp
from jax import lax
from jax.experimental import pallas as pl
from jax.experimental.pallas import tpu as pltpu
from jax.experimental.pallas import tpu_sc as plsc

D_MODEL = 768
LANES = 16
VPR = D_MODEL // LANES
NC = 2
NS = 16
NW = NC * NS
POS_PER_CHUNK = 8
REG_BLOCK = 16
NRING = 4
GAHEAD = NRING - 1
PRING = 2


@functools.partial(jax.jit, static_argnums=(3, 4))
def _embed(xr, token_emb, pos_emb, batch, seq_len):
    s_per_w = seq_len // NW
    n_chunks = s_per_w // POS_PER_CHUNK
    chunk_rows = POS_PER_CHUNK * batch
    mesh = plsc.VectorSubcoreMesh(core_axis_name="c", subcore_axis_name="s")

    @functools.partial(
        pl.kernel,
        out_type=jax.ShapeDtypeStruct((batch, seq_len, D_MODEL), jnp.float32),
        mesh=mesh,
        scratch_types=[
            pltpu.VMEM((n_chunks, chunk_rows), jnp.int32),
            pltpu.VMEM((PRING, POS_PER_CHUNK, D_MODEL), jnp.float32),
            *[pltpu.VMEM((chunk_rows, D_MODEL), jnp.float32)
              for _ in range(NRING)],
            *[pltpu.SemaphoreType.DMA for _ in range(2 * NRING + PRING)],
        ],
    )
    def body(xr_hbm, tok_hbm, pos_hbm, out_hbm, idx_v, pos_ring, *rest):
        bufs = rest[:NRING]
        gsems = rest[NRING:2 * NRING]
        ssems = rest[2 * NRING:3 * NRING]
        psems = rest[3 * NRING:]
        wid = lax.axis_index("s") * NC + lax.axis_index("c")
        s_base = wid * s_per_w

        pltpu.sync_copy(xr_hbm.at[wid], idx_v)

        def start_prefill(c):
            j2 = c % PRING
            return pltpu.async_copy(
                pos_hbm.at[pl.ds(s_base + c * POS_PER_CHUNK, POS_PER_CHUNK)],
                pos_ring.at[j2], psems[j2])

        def start_gather(c):
            j = c % NRING
            return pltpu.async_copy(
                tok_hbm.at[idx_v.at[c]], bufs[j], gsems[j])

        def start_stores(c):
            j = c % NRING
            return [
                pltpu.async_copy(
                    bufs[j].at[pl.ds(b * POS_PER_CHUNK, POS_PER_CHUNK)],
                    out_hbm.at[b, pl.ds(s_base + c * POS_PER_CHUNK,
                                        POS_PER_CHUNK)],
                    ssems[j],
                )
                for b in range(batch)
            ]

        gathers = [None] * NRING
        stores = [None] * NRING
        prefills = [None] * PRING
        for c in range(PRING):
            prefills[c] = start_prefill(c)
        for c in range(GAHEAD):
            gathers[c] = start_gather(c)
        for c in range(n_chunks):
            j = c % NRING
            j2 = c % PRING
            gathers[j].wait()
            prefills[j2].wait()

            buf = bufs[j]

            def add_pos(p, _):
                def t_body(t, _):
                    base = t * (REG_BLOCK * LANES)
                    regs = [
                        pos_ring[j2, p, pl.ds(base + v * LANES, LANES)]
                        for v in range(REG_BLOCK)
                    ]
                    for b in range(batch):
                        r = b * POS_PER_CHUNK + p
                        for v in range(REG_BLOCK):
                            sl = pl.ds(base + v * LANES, LANES)
                            plsc.addupdate(buf.at[r, sl], regs[v])
                    return ()

                return lax.fori_loop(0, VPR // REG_BLOCK, t_body, (),
                                     unroll=False)

            lax.fori_loop(0, POS_PER_CHUNK, add_pos, (), unroll=False)
            stores[j] = start_stores(c)
            if c + PRING < n_chunks:
                prefills[j2] = start_prefill(c + PRING)
            if c + GAHEAD < n_chunks:
                k = (c + GAHEAD) % NRING
                if stores[k] is not None:
                    for hnd in stores[k]:
                        hnd.wait()
                gathers[k] = start_gather(c + GAHEAD)
        for sset in stores:
            if sset is not None:
                for hnd in sset:
                    hnd.wait()

    return body(xr, token_emb, pos_emb)


def kernel(x, token_emb, pos_emb):
    batch, seq = x.shape
    s_per_w = seq // NW
    n_chunks = s_per_w // POS_PER_CHUNK
    xr = (x.astype(jnp.int32)
          .reshape(batch, NW, n_chunks, POS_PER_CHUNK)
          .transpose(1, 2, 0, 3)
          .reshape(NW, n_chunks, batch * POS_PER_CHUNK))
    return _embed(xr, token_emb, pos_emb, batch, seq)


# speedup vs baseline: 1.1505x; 1.0269x over previous
import functools

import jax
import jax.numpy as jnp
from jax import lax
from jax.experimental import pallas as pl
from jax.experimental.pallas import tpu as pltpu
from jax.experimental.pallas import tpu_sc as plsc

D_MODEL = 768
LANES = 16
VPR = D_MODEL // LANES
NC = 2
NS = 16
NW = NC * NS
POS_PER_CHUNK = 8
REG_BLOCK = 8
NRING = 4
GAHEAD = NRING - 1
PRING = 2


@functools.partial(jax.jit, static_argnums=(3, 4))
def _embed(xr, token_emb, pos_emb, batch, seq_len):
    s_per_w = seq_len // NW
    n_chunks = s_per_w // POS_PER_CHUNK
    chunk_rows = POS_PER_CHUNK * batch
    mesh = plsc.VectorSubcoreMesh(core_axis_name="c", subcore_axis_name="s")

    @functools.partial(
        pl.kernel,
        out_type=jax.ShapeDtypeStruct((batch, seq_len, D_MODEL), jnp.float32),
        mesh=mesh,
        scratch_types=[
            pltpu.VMEM((n_chunks, chunk_rows), jnp.int32),
            pltpu.VMEM((PRING, POS_PER_CHUNK, D_MODEL), jnp.float32),
            *[pltpu.VMEM((chunk_rows, D_MODEL), jnp.float32)
              for _ in range(NRING)],
            *[pltpu.SemaphoreType.DMA for _ in range(2 * NRING + PRING)],
        ],
    )
    def body(xr_hbm, tok_hbm, pos_hbm, out_hbm, idx_v, pos_ring, *rest):
        bufs = rest[:NRING]
        gsems = rest[NRING:2 * NRING]
        ssems = rest[2 * NRING:3 * NRING]
        psems = rest[3 * NRING:]
        wid = lax.axis_index("s") * NC + lax.axis_index("c")
        s_base = wid * s_per_w

        pltpu.sync_copy(xr_hbm.at[wid], idx_v)

        def start_prefill(c):
            j2 = c % PRING
            return pltpu.async_copy(
                pos_hbm.at[pl.ds(s_base + c * POS_PER_CHUNK, POS_PER_CHUNK)],
                pos_ring.at[j2], psems[j2])

        def start_gather(c):
            j = c % NRING
            return pltpu.async_copy(
                tok_hbm.at[idx_v.at[c]], bufs[j], gsems[j])

        def start_stores(c):
            j = c % NRING
            return [
                pltpu.async_copy(
                    bufs[j].at[pl.ds(b * POS_PER_CHUNK, POS_PER_CHUNK)],
                    out_hbm.at[b, pl.ds(s_base + c * POS_PER_CHUNK,
                                        POS_PER_CHUNK)],
                    ssems[j],
                )
                for b in range(batch)
            ]

        gathers = [None] * NRING
        stores = [None] * NRING
        prefills = [None] * PRING
        for c in range(PRING):
            prefills[c] = start_prefill(c)
        for c in range(GAHEAD):
            gathers[c] = start_gather(c)
        for c in range(n_chunks):
            j = c % NRING
            j2 = c % PRING
            gathers[j].wait()
            prefills[j2].wait()

            buf = bufs[j]

            def add_pos(p, _):
                def t_body(t, _):
                    base = t * (REG_BLOCK * LANES)
                    regs = [
                        pos_ring[j2, p, pl.ds(base + v * LANES, LANES)]
                        for v in range(REG_BLOCK)
                    ]
                    for b in range(batch):
                        r = b * POS_PER_CHUNK + p
                        for v in range(REG_BLOCK):
                            sl = pl.ds(base + v * LANES, LANES)
                            plsc.addupdate(buf.at[r, sl], regs[v])
                    return ()

                return lax.fori_loop(0, VPR // REG_BLOCK, t_body, (),
                                     unroll=False)

            lax.fori_loop(0, POS_PER_CHUNK, add_pos, (), unroll=False)
            stores[j] = start_stores(c)
            if c + PRING < n_chunks:
                prefills[j2] = start_prefill(c + PRING)
            if c + GAHEAD < n_chunks:
                k = (c + GAHEAD) % NRING
                if stores[k] is not None:
                    for hnd in stores[k]:
                        hnd.wait()
                gathers[k] = start_gather(c + GAHEAD)
        for sset in stores:
            if sset is not None:
                for hnd in sset:
                    hnd.wait()

    return body(xr, token_emb, pos_emb)


def kernel(x, token_emb, pos_emb):
    batch, seq = x.shape
    s_per_w = seq // NW
    n_chunks = s_per_w // POS_PER_CHUNK
    xr = (x.astype(jnp.int32)
          .reshape(batch, NW, n_chunks, POS_PER_CHUNK)
          .transpose(1, 2, 0, 3)
          .reshape(NW, n_chunks, batch * POS_PER_CHUNK))
    return _embed(xr, token_emb, pos_emb, batch, seq)


# speedup vs baseline: 1.1655x; 1.0130x over previous
import functools

import jax
import jax.numpy as jnp
from jax import lax
from jax.experimental import pallas as pl
from jax.experimental.pallas import tpu as pltpu
from jax.experimental.pallas import tpu_sc as plsc

D_MODEL = 768
LANES = 16
VPR = D_MODEL // LANES
NC = 2
NS = 16
NW = NC * NS
POS_PER_CHUNK = 8
REG_BLOCK = 4
NRING = 4
GAHEAD = NRING - 1
PRING = 2


@functools.partial(jax.jit, static_argnums=(3, 4))
def _embed(xr, token_emb, pos_emb, batch, seq_len):
    s_per_w = seq_len // NW
    n_chunks = s_per_w // POS_PER_CHUNK
    chunk_rows = POS_PER_CHUNK * batch
    mesh = plsc.VectorSubcoreMesh(core_axis_name="c", subcore_axis_name="s")

    @functools.partial(
        pl.kernel,
        out_type=jax.ShapeDtypeStruct((batch, seq_len, D_MODEL), jnp.float32),
        mesh=mesh,
        scratch_types=[
            pltpu.VMEM((n_chunks, chunk_rows), jnp.int32),
            pltpu.VMEM((PRING, POS_PER_CHUNK, D_MODEL), jnp.float32),
            *[pltpu.VMEM((chunk_rows, D_MODEL), jnp.float32)
              for _ in range(NRING)],
            *[pltpu.SemaphoreType.DMA for _ in range(2 * NRING + PRING)],
        ],
    )
    def body(xr_hbm, tok_hbm, pos_hbm, out_hbm, idx_v, pos_ring, *rest):
        bufs = rest[:NRING]
        gsems = rest[NRING:2 * NRING]
        ssems = rest[2 * NRING:3 * NRING]
        psems = rest[3 * NRING:]
        wid = lax.axis_index("s") * NC + lax.axis_index("c")
        s_base = wid * s_per_w

        pltpu.sync_copy(xr_hbm.at[wid], idx_v)

        def start_prefill(c):
            j2 = c % PRING
            return pltpu.async_copy(
                pos_hbm.at[pl.ds(s_base + c * POS_PER_CHUNK, POS_PER_CHUNK)],
                pos_ring.at[j2], psems[j2])

        def start_gather(c):
            j = c % NRING
            return pltpu.async_copy(
                tok_hbm.at[idx_v.at[c]], bufs[j], gsems[j])

        def start_stores(c):
            j = c % NRING
            return [
                pltpu.async_copy(
                    bufs[j].at[pl.ds(b * POS_PER_CHUNK, POS_PER_CHUNK)],
                    out_hbm.at[b, pl.ds(s_base + c * POS_PER_CHUNK,
                                        POS_PER_CHUNK)],
                    ssems[j],
                )
                for b in range(batch)
            ]

        gathers = [None] * NRING
        stores = [None] * NRING
        prefills = [None] * PRING
        for c in range(PRING):
            prefills[c] = start_prefill(c)
        for c in range(GAHEAD):
            gathers[c] = start_gather(c)
        for c in range(n_chunks):
            j = c % NRING
            j2 = c % PRING
            gathers[j].wait()
            prefills[j2].wait()

            buf = bufs[j]

            def add_pos(p, _):
                def t_body(t, _):
                    base = t * (REG_BLOCK * LANES)
                    regs = [
                        pos_ring[j2, p, pl.ds(base + v * LANES, LANES)]
                        for v in range(REG_BLOCK)
                    ]
                    for b in range(batch):
                        r = b * POS_PER_CHUNK + p
                        for v in range(REG_BLOCK):
                            sl = pl.ds(base + v * LANES, LANES)
                            plsc.addupdate(buf.at[r, sl], regs[v])
                    return ()

                return lax.fori_loop(0, VPR // REG_BLOCK, t_body, (),
                                     unroll=False)

            lax.fori_loop(0, POS_PER_CHUNK, add_pos, (), unroll=False)
            stores[j] = start_stores(c)
            if c + PRING < n_chunks:
                prefills[j2] = start_prefill(c + PRING)
            if c + GAHEAD < n_chunks:
                k = (c + GAHEAD) % NRING
                if stores[k] is not None:
                    for hnd in stores[k]:
                        hnd.wait()
                gathers[k] = start_gather(c + GAHEAD)
        for sset in stores:
            if sset is not None:
                for hnd in sset:
                    hnd.wait()

    return body(xr, token_emb, pos_emb)


def kernel(x, token_emb, pos_emb):
    batch, seq = x.shape
    s_per_w = seq // NW
    n_chunks = s_per_w // POS_PER_CHUNK
    xr = (x.astype(jnp.int32)
          .reshape(batch, NW, n_chunks, POS_PER_CHUNK)
          .transpose(1, 2, 0, 3)
          .reshape(NW, n_chunks, batch * POS_PER_CHUNK))
    return _embed(xr, token_emb, pos_emb, batch, seq)
